# baseline (speedup 1.0000x reference)
import functools

import jax
import jax.numpy as jnp
from jax import lax
from jax.experimental import pallas as pl
from jax.experimental.pallas import tpu as pltpu
from jax.experimental.pallas import tpu_sc as plsc

_B = 16384
_D = 64
_OUT_D = 76
_V = 100001
_VMAIN = 99968
_VTAIL = _V - _VMAIN
_VPAD = _VMAIN + 128
_CHUNK = 4096
_NCHUNK = _B // _CHUNK


_HALF = 50048
_UMAIN = _VMAIN - _HALF
_USZ = _UMAIN + 128
_UCH = 8192


def _body(uid_h, feats_h, sex_h, tableT_h, tail_h, consts_h, sexf_h,
          outT_h, uidc_v, low_v, upp_v, ocol_v, consts_v, sexf_v,
          seml, semu, osem):
    c = lax.axis_index("c")
    s = lax.axis_index("s")
    wid = s * 2 + c

    pltpu.sync_copy(consts_h, consts_v)
    pltpu.sync_copy(sexf_h, sexf_v)

    def fire_low(col):
        cp = pltpu.make_async_copy(
            tableT_h.at[col, pl.ds(0, _HALF)], low_v, seml)
        cp.start()
        return (cp,)

    def fire_upp(col):
        c1 = pltpu.make_async_copy(
            tableT_h.at[col, pl.ds(_HALF, _UMAIN)],
            upp_v.at[pl.ds(0, _UMAIN)], semu)
        c1.start()
        c2 = pltpu.make_async_copy(
            tail_h.at[col], upp_v.at[pl.ds(_UMAIN, 128)], semu)
        c2.start()
        return (c1, c2)

    cpl = fire_low(2 * wid)
    cpu = fire_upp(2 * wid)
    pending = []

    for slot in range(2):
        col = 2 * wid + slot
        for cp in cpl:
            cp.wait()
        for cp in pending:
            cp.wait()
        pending = []

        for k in range(_B // _UCH):
            pltpu.sync_copy(uid_h.at[pl.ds(k * _UCH, _UCH)], uidc_v)

            def lblk(i, carry):
                for u in range(8):
                    off = i * 128 + u * 16
                    idx = uidc_v[pl.ds(off, 16)]
                    vals = plsc.load_gather(low_v, [jnp.minimum(idx, _HALF - 1)])
                    ocol_v[pl.ds(k * _UCH + off, 16)] = vals
                return carry
            lax.fori_loop(0, _UCH // 128, lblk, 0)
        if slot == 0:
            nxt_low = fire_low(col + 1)

        for cp in cpu:
            cp.wait()
        for k in range(_B // _UCH):
            pltpu.sync_copy(uid_h.at[pl.ds(k * _UCH, _UCH)], uidc_v)

            def ublk(i, carry):
                for u in range(8):
                    off = i * 128 + u * 16
                    pos = k * _UCH + off
                    idx = uidc_v[pl.ds(off, 16)]
                    idxu = jnp.maximum(idx - _HALF, 0)
                    vals = plsc.load_gather(upp_v, [idxu])
                    prev = ocol_v[pl.ds(pos, 16)]
                    ocol_v[pl.ds(pos, 16)] = jnp.where(idx >= _HALF, vals, prev)
                return carry
            lax.fori_loop(0, _UCH // 128, ublk, 0)
        if slot == 0:
            cpu = fire_upp(col + 1)
            cpl = nxt_low

        for q in range(4):
            cp = pltpu.make_async_copy(
                ocol_v.at[pl.ds(q * _CHUNK, _CHUNK)],
                outT_h.at[col, pl.ds(q * _CHUNK, _CHUNK)], osem)
            cp.start()
            pending.append(cp)

    for cp in pending:
        cp.wait()

    tcol = _D + wid

    @pl.when(wid < 4)
    def _feat():
        mean = consts_v[pl.ds(16 * wid, 16)]
        scale = consts_v[pl.ds(_D + 16 * wid, 16)]
        pltpu.sync_copy(feats_h.at[wid], low_v.at[pl.ds(0, _B)])
        def fblk(i, carry):
            for u in range(8):
                off = i * 128 + u * 16
                x = low_v[pl.ds(off, 16)]
                ocol_v[pl.ds(off, 16)] = (x - mean) * scale
            return carry
        lax.fori_loop(0, _B // 128, fblk, 0)
        pltpu.sync_copy(ocol_v, outT_h.at[tcol])

    @pl.when((wid >= 4) & (wid < 12))
    def _sex():
        j = tcol - (_D + 4)
        for k in range(_B // _UCH):
            pltpu.sync_copy(sex_h.at[pl.ds(k * _UCH, _UCH)], uidc_v)
            def sblk(i, carry):
                for u in range(8):
                    off = i * 128 + u * 16
                    sv = uidc_v[pl.ds(off, 16)]
                    vals = plsc.load_gather(sexf_v, [sv * 8 + j])
                    ocol_v[pl.ds(k * _UCH + off, 16)] = vals
                return carry
            lax.fori_loop(0, _UCH // 128, sblk, 0)
        pltpu.sync_copy(ocol_v, outT_h.at[tcol])


@jax.jit
def _run(uid, feats, sex, tableT, tail, consts, sexf):
    mesh = plsc.VectorSubcoreMesh(core_axis_name="c", subcore_axis_name="s")
    f = functools.partial(
        pl.kernel,
        out_type=jax.ShapeDtypeStruct((_OUT_D, _B), jnp.float32),
        mesh=mesh,
        compiler_params=pltpu.CompilerParams(
            needs_layout_passes=False, use_tc_tiling_on_sc=True),
        scratch_types=[
            pltpu.VMEM((_UCH,), jnp.int32),
            pltpu.VMEM((_HALF,), jnp.float32),
            pltpu.VMEM((_USZ,), jnp.float32),
            pltpu.VMEM((_B,), jnp.float32),
            pltpu.VMEM((128,), jnp.float32),
            pltpu.VMEM((128,), jnp.float32),
            pltpu.SemaphoreType.DMA,
            pltpu.SemaphoreType.DMA,
            pltpu.SemaphoreType.DMA,
        ],
    )(_body)
    return f(uid, feats, sex, tableT, tail, consts, sexf)


def kernel(User_ID, Age, Body_Weight, Body_Height, Cal_Need, sex, user_table,
           sex_table, feat_mean, feat_var):
    scale = 1.0 / jnp.maximum(jnp.sqrt(feat_var), 1e-7)
    consts = jnp.concatenate(
        [jnp.broadcast_to(feat_mean[:, None], (4, 16)).reshape(-1),
         jnp.broadcast_to(scale[:, None], (4, 16)).reshape(-1)])
    feats = jnp.stack([Age, Body_Weight, Body_Height, Cal_Need])
    tableT = user_table.T
    tail = jnp.pad(user_table[_VMAIN:].T, ((0, 0), (0, 128 - _VTAIL)))
    sexf = jnp.pad(sex_table.reshape(-1), (0, 128 - 24))
    outT = _run(User_ID.astype(jnp.int32), feats, sex.astype(jnp.int32),
                tableT, tail, consts, sexf)
    return outT.T


# speedup vs baseline: 1.4377x; 1.4377x over previous
import functools

import jax
import jax.numpy as jnp
from jax import lax
from jax.experimental import pallas as pl
from jax.experimental.pallas import tpu as pltpu
from jax.experimental.pallas import tpu_sc as plsc

_B = 16384
_D = 64
_OUT_D = 76
_V = 100001
_VMAIN = 99968
_VTAIL = _V - _VMAIN
_VPAD = _VMAIN + 128
_CHUNK = 4096
_NCHUNK = _B // _CHUNK


def _body(uid_h, age_h, bw_h, bh_h, cal_h, sex_h, tableT_h, tail_h, parms_h,
          outT_h, uid_v, colbuf_v, ocol0_v, ocol1_v, parms_v,
          sem, osem0, osem1):
    c = lax.axis_index("c")
    s = lax.axis_index("s")
    wid = s * 2 + c

    pltpu.sync_copy(parms_h, parms_v)
    pltpu.sync_copy(uid_h, uid_v)

    ocols = (ocol0_v, ocol1_v)
    osems = (osem0, osem1)
    pending = [None, None]

    def out_write(k, col):
        buf = ocols[k % 2]
        cp = pltpu.make_async_copy(
            buf, outT_h.at[col, pl.ds(k * _CHUNK, _CHUNK)], osems[k % 2])
        cp.start()
        pending[k % 2] = cp

    def drain(k):
        if pending[k % 2] is not None:
            pending[k % 2].wait()
            pending[k % 2] = None

    def gather_chunk(k, col):
        drain(k)
        buf = ocols[k % 2]

        def blk(i, carry):
            for u in range(8):
                off = i * 128 + u * 16
                idx = uid_v[pl.ds(k * _CHUNK + off, 16)]
                vals = plsc.load_gather(colbuf_v, [idx])
                buf[pl.ds(off, 16)] = vals
            return carry
        lax.fori_loop(0, _CHUNK // 128, blk, 0)
        out_write(k, col)

    _PARTS = (0, 25088, 50176, 75264, _VMAIN)
    for slot in range(2):
        col = 2 * wid + slot
        cps = []
        for p in range(4):
            lo, hi = _PARTS[p], _PARTS[p + 1]
            cp = pltpu.make_async_copy(
                tableT_h.at[col, pl.ds(lo, hi - lo)],
                colbuf_v.at[pl.ds(lo, hi - lo)], sem)
            cp.start()
            cps.append(cp)
        cp = pltpu.make_async_copy(
            tail_h.at[col], colbuf_v.at[pl.ds(_VMAIN, 128)], sem)
        cp.start()
        cps.append(cp)
        for cp in cps:
            cp.wait()
        for k in range(_NCHUNK):
            gather_chunk(k, col)

    drain(0)
    drain(1)

    tcol = _D + wid

    @pl.when(wid < 4)
    def _feat():
        mean = parms_v[pl.ds(16 * wid, 16)]
        scale = parms_v[pl.ds(_D + 16 * wid, 16)]
        for f, ref in enumerate((age_h, bw_h, bh_h, cal_h)):
            @pl.when(wid == f)
            def _stage(ref=ref):
                pltpu.sync_copy(ref, colbuf_v.at[pl.ds(0, _B)])
        for k in range(_NCHUNK):
            def fblk(i, carry):
                for u in range(8):
                    off = i * 128 + u * 16
                    x = colbuf_v[pl.ds(k * _CHUNK + off, 16)]
                    ocol0_v[pl.ds(off, 16)] = (x - mean) * scale
                return carry
            lax.fori_loop(0, _CHUNK // 128, fblk, 0)
            pltpu.sync_copy(ocol0_v, outT_h.at[tcol, pl.ds(k * _CHUNK, _CHUNK)])

    @pl.when((wid >= 4) & (wid < 12))
    def _sex():
        j = tcol - (_D + 4)
        pltpu.sync_copy(sex_h, uid_v)
        for k in range(_NCHUNK):
            def sblk(i, carry):
                for u in range(8):
                    off = i * 128 + u * 16
                    sv = uid_v[pl.ds(k * _CHUNK + off, 16)]
                    vals = plsc.load_gather(parms_v, [128 + sv * 8 + j])
                    ocol0_v[pl.ds(off, 16)] = vals
                return carry
            lax.fori_loop(0, _CHUNK // 128, sblk, 0)
            pltpu.sync_copy(ocol0_v, outT_h.at[tcol, pl.ds(k * _CHUNK, _CHUNK)])


@jax.jit
def _run(uid, age, bw, bh, cal, sex, tableT, tail, parms):
    mesh = plsc.VectorSubcoreMesh(core_axis_name="c", subcore_axis_name="s")
    f = functools.partial(
        pl.kernel,
        out_type=jax.ShapeDtypeStruct((_OUT_D, _B), jnp.float32),
        mesh=mesh,
        compiler_params=pltpu.CompilerParams(
            needs_layout_passes=False, use_tc_tiling_on_sc=True),
        scratch_types=[
            pltpu.VMEM((_B,), jnp.int32),
            pltpu.VMEM((_VPAD,), jnp.float32),
            pltpu.VMEM((_CHUNK,), jnp.float32),
            pltpu.VMEM((_CHUNK,), jnp.float32),
            pltpu.VMEM((256,), jnp.float32),
            pltpu.SemaphoreType.DMA,
            pltpu.SemaphoreType.DMA,
            pltpu.SemaphoreType.DMA,
        ],
    )(_body)
    return f(uid, age, bw, bh, cal, sex, tableT, tail, parms)


def kernel(User_ID, Age, Body_Weight, Body_Height, Cal_Need, sex, user_table,
           sex_table, feat_mean, feat_var):
    scale = 1.0 / jnp.maximum(jnp.sqrt(feat_var), 1e-7)
    parms = jnp.concatenate(
        [jnp.broadcast_to(feat_mean[:, None], (4, 16)).reshape(-1),
         jnp.broadcast_to(scale[:, None], (4, 16)).reshape(-1),
         jnp.pad(sex_table.reshape(-1), (0, 128 - 24))])
    tableT = user_table.T
    tail = jnp.pad(user_table[_VMAIN:].T, ((0, 0), (0, 128 - _VTAIL)))
    outT = _run(User_ID.astype(jnp.int32), Age, Body_Weight, Body_Height,
                Cal_Need, sex.astype(jnp.int32), tableT, tail, parms)
    return outT.T


# speedup vs baseline: 1.5166x; 1.0549x over previous
import functools

import jax
import jax.numpy as jnp
from jax import lax
from jax.experimental import pallas as pl
from jax.experimental.pallas import tpu as pltpu
from jax.experimental.pallas import tpu_sc as plsc

_B = 16384
_D = 64
_OUT_D = 76
_V = 100001
_VMAIN = 99968
_VTAIL = _V - _VMAIN
_VPAD = _VMAIN + 128
_CHUNK = 4096
_NCHUNK = _B // _CHUNK


def _body(uid_h, age_h, bw_h, bh_h, cal_h, sex_h, tableT_h, tail_h, parms_h,
          outT_h, uid_v, colbuf_v, ocol0_v, ocol1_v, parms_v,
          sem, osem0, osem1):
    c = lax.axis_index("c")
    s = lax.axis_index("s")
    wid = s * 2 + c

    pltpu.sync_copy(parms_h, parms_v)
    pltpu.sync_copy(uid_h, uid_v)

    ocols = (ocol0_v, ocol1_v)
    osems = (osem0, osem1)
    pending = [None, None]

    def out_write(k, col):
        buf = ocols[k % 2]
        cp = pltpu.make_async_copy(
            buf, outT_h.at[col, pl.ds(k * _CHUNK, _CHUNK)], osems[k % 2])
        cp.start()
        pending[k % 2] = cp

    def drain(k):
        if pending[k % 2] is not None:
            pending[k % 2].wait()
            pending[k % 2] = None

    def gather_chunk(k, col):
        drain(k)
        buf = ocols[k % 2]

        def blk(i, carry):
            for u in range(16):
                off = i * 256 + u * 16
                idx = uid_v[pl.ds(k * _CHUNK + off, 16)]
                vals = plsc.load_gather(colbuf_v, [idx])
                buf[pl.ds(off, 16)] = vals
            return carry
        lax.fori_loop(0, _CHUNK // 256, blk, 0)
        out_write(k, col)

    _PARTS = (0, 25088, 50176, 75264, _VMAIN)
    for slot in range(2):
        col = 2 * wid + slot
        cps = []
        for p in range(4):
            lo, hi = _PARTS[p], _PARTS[p + 1]
            cp = pltpu.make_async_copy(
                tableT_h.at[col, pl.ds(lo, hi - lo)],
                colbuf_v.at[pl.ds(lo, hi - lo)], sem)
            cp.start()
            cps.append(cp)
        cp = pltpu.make_async_copy(
            tail_h.at[col], colbuf_v.at[pl.ds(_VMAIN, 128)], sem)
        cp.start()
        cps.append(cp)
        for cp in cps:
            cp.wait()
        for k in range(_NCHUNK):
            gather_chunk(k, col)

    drain(0)
    drain(1)

    _HB = _B // 2
    fsel = wid - jnp.where(wid >= 12, 12, 0)
    tcol = _D + fsel
    hbase = jnp.where(wid >= 12, _HB, 0)

    @pl.when((wid < 24) & (fsel < 4))
    def _feat():
        mean = parms_v[pl.ds(16 * fsel, 16)]
        scale = parms_v[pl.ds(_D + 16 * fsel, 16)]
        for f, ref in enumerate((age_h, bw_h, bh_h, cal_h)):
            @pl.when(fsel == f)
            def _stage(ref=ref):
                pltpu.sync_copy(ref.at[pl.ds(hbase, _HB)],
                                colbuf_v.at[pl.ds(0, _HB)])
        for k in range(_HB // _CHUNK):
            def fblk(i, carry):
                for u in range(16):
                    off = i * 256 + u * 16
                    x = colbuf_v[pl.ds(k * _CHUNK + off, 16)]
                    ocol0_v[pl.ds(off, 16)] = (x - mean) * scale
                return carry
            lax.fori_loop(0, _CHUNK // 256, fblk, 0)
            pltpu.sync_copy(
                ocol0_v, outT_h.at[tcol, pl.ds(hbase + k * _CHUNK, _CHUNK)])

    @pl.when((wid < 24) & (fsel >= 4))
    def _sex():
        j = fsel - 4
        pltpu.sync_copy(sex_h.at[pl.ds(hbase, _HB)], uid_v.at[pl.ds(0, _HB)])
        for k in range(_HB // _CHUNK):
            def sblk(i, carry):
                for u in range(16):
                    off = i * 256 + u * 16
                    sv = uid_v[pl.ds(k * _CHUNK + off, 16)]
                    vals = plsc.load_gather(parms_v, [128 + sv * 8 + j])
                    ocol0_v[pl.ds(off, 16)] = vals
                return carry
            lax.fori_loop(0, _CHUNK // 256, sblk, 0)
            pltpu.sync_copy(
                ocol0_v, outT_h.at[tcol, pl.ds(hbase + k * _CHUNK, _CHUNK)])


@jax.jit
def _run(uid, age, bw, bh, cal, sex, tableT, tail, parms):
    mesh = plsc.VectorSubcoreMesh(core_axis_name="c", subcore_axis_name="s")
    f = functools.partial(
        pl.kernel,
        out_type=jax.ShapeDtypeStruct((_OUT_D, _B), jnp.float32),
        mesh=mesh,
        compiler_params=pltpu.CompilerParams(
            needs_layout_passes=False, use_tc_tiling_on_sc=True),
        scratch_types=[
            pltpu.VMEM((_B,), jnp.int32),
            pltpu.VMEM((_VPAD,), jnp.float32),
            pltpu.VMEM((_CHUNK,), jnp.float32),
            pltpu.VMEM((_CHUNK,), jnp.float32),
            pltpu.VMEM((256,), jnp.float32),
            pltpu.SemaphoreType.DMA,
            pltpu.SemaphoreType.DMA,
            pltpu.SemaphoreType.DMA,
        ],
    )(_body)
    return f(uid, age, bw, bh, cal, sex, tableT, tail, parms)


def kernel(User_ID, Age, Body_Weight, Body_Height, Cal_Need, sex, user_table,
           sex_table, feat_mean, feat_var):
    scale = 1.0 / jnp.maximum(jnp.sqrt(feat_var), 1e-7)
    parms = jnp.concatenate(
        [jnp.broadcast_to(feat_mean[:, None], (4, 16)).reshape(-1),
         jnp.broadcast_to(scale[:, None], (4, 16)).reshape(-1),
         jnp.pad(sex_table.reshape(-1), (0, 128 - 24))])
    tableT = user_table.T
    tail = jnp.pad(user_table[_VMAIN:].T, ((0, 0), (0, 128 - _VTAIL)))
    outT = _run(User_ID.astype(jnp.int32), Age, Body_Weight, Body_Height,
                Cal_Need, sex.astype(jnp.int32), tableT, tail, parms)
    return outT.T
